# baseline (speedup 1.0000x reference)
import functools

import jax
import jax.numpy as jnp
import numpy as np
from jax import lax
from jax.experimental import pallas as pl
from jax.experimental.pallas import tpu as pltpu
from jax.experimental.pallas import tpu_sc as plsc

N = 100000
E = 6400000
B = 64
K = 5
D_IN = 3
H1 = 1024
H2 = 512
H3 = 256
CLS = 40
EPS = 1e-5

NC = 2
NS = 16
NW = NC * NS
NPAD = 100352
SL = NPAD // NS
SLW = NPAD // NW
CH = 6400
NCHUNK = 32
EPW = CH * NCHUNK
EPAD = EPW * NW

@functools.cache
def _mesh():
    return plsc.VectorSubcoreMesh(
        core_axis_name="c", subcore_axis_name="s",
        num_cores=NC, num_subcores=NS)


_f32 = jnp.float32
_i32 = jnp.int32


def _I(v):
    return jnp.int32(v)


_c16 = 16
_cNS = NS
_cSL = SL
_cSLW = SLW
_cCH = CH
_cEPW = EPW


def _deg_body(col_hbm, batch_hbm, degA, degB, cntA, cntB,
              colv, batv, onesv, zbuf, degS, cntS, sem):
    c = lax.axis_index("c")
    s = lax.axis_index("s")
    wid = c * _cNS + s
    sl = pl.ds(s * _cSL, SL)

    def _fz(i, _):
        zbuf[pl.ds(i * _c16, 16)] = jnp.zeros((16,), _f32)
        return np.int32(0)
    lax.fori_loop(np.int32(0), np.int32(SL // 16), _fz, np.int32(0))

    def _fo(i, _):
        onesv[pl.ds(i * _c16, 16)] = jnp.ones((16,), _f32)
        return np.int32(0)
    lax.fori_loop(np.int32(0), np.int32(CH // 16), _fo, np.int32(0))

    pltpu.sync_copy(zbuf.at[pl.ds(0, SL)], degS.at[sl])

    @pl.when(s == 0)
    def _():
        pltpu.sync_copy(zbuf.at[pl.ds(0, 128)], cntS)

    plsc.subcore_barrier()

    def _chunk(k, _):
        off = wid * _cEPW + k * _cCH
        pltpu.sync_copy(col_hbm.at[pl.ds(off, CH)], colv)
        pltpu.sync_copy(onesv, degS.at[colv], add=True)
        return np.int32(0)
    lax.fori_loop(np.int32(0), np.int32(NCHUNK), _chunk, np.int32(0))

    pltpu.sync_copy(batch_hbm.at[pl.ds(wid * _cSLW, SLW)], batv)
    pltpu.sync_copy(onesv.at[pl.ds(0, SLW)], cntS.at[batv], add=True)

    plsc.subcore_barrier()

    @pl.when(c == 0)
    def _():
        pltpu.sync_copy(degS.at[sl], degA.at[sl])

        @pl.when(s == 0)
        def _():
            pltpu.sync_copy(cntS, cntA)

    @pl.when(c == 1)
    def _():
        pltpu.sync_copy(degS.at[sl], degB.at[sl])

        @pl.when(s == 0)
        def _():
            pltpu.sync_copy(cntS, cntB)


@functools.cache
def _deg_kernel():
  return pl.kernel(
    _deg_body,
    out_type=(
        jax.ShapeDtypeStruct((NPAD,), _f32),
        jax.ShapeDtypeStruct((NPAD,), _f32),
        jax.ShapeDtypeStruct((128,), _f32),
        jax.ShapeDtypeStruct((128,), _f32),
    ),
    mesh=_mesh(),
    scratch_types=(
        pltpu.VMEM((CH,), _i32),
        pltpu.VMEM((SLW,), _i32),
        pltpu.VMEM((CH,), _f32),
        pltpu.VMEM((SL,), _f32),
        pltpu.VMEM_SHARED((NPAD,), _f32),
        pltpu.VMEM_SHARED((128,), _f32),
        pltpu.SemaphoreType.DMA,
    ),
  )


def _hop_body(row_hbm, col_hbm, w_hbm,
              aA0, aA1, aA2, aB0, aB1, aB2,
              oA0, oA1, oA2, oB0, oB1, oB2,
              rowv0, colv0, rowv1, colv1,
              gva0, gva1, gva2, gvb0, gvb1, gvb2,
              bufA, bufB, bufW, bufO,
              zS0, zS1, zS2, aS0, aS1, aS2, sem):
    c = lax.axis_index("c")
    s = lax.axis_index("s")
    wid = c * _cNS + s
    sl = pl.ds(s * _cSL, SL)
    aAs = (aA0, aA1, aA2)
    aBs = (aB0, aB1, aB2)
    oAs = (oA0, oA1, oA2)
    oBs = (oB0, oB1, oB2)
    zS = (zS0, zS1, zS2)
    aS = (aS0, aS1, aS2)

    pltpu.sync_copy(w_hbm.at[sl], bufW)
    for f in range(3):
        pltpu.sync_copy(aAs[f].at[sl], bufA)
        pltpu.sync_copy(aBs[f].at[sl], bufB)

        def _cb(i, _):
            ix = pl.ds(i * _c16, 16)
            bufO[ix] = (bufA[ix] + bufB[ix]) * bufW[ix]
            bufA[ix] = jnp.zeros((16,), _f32)
            return np.int32(0)
        lax.fori_loop(np.int32(0), np.int32(SL // 16), _cb, np.int32(0))

        pltpu.sync_copy(bufO, zS[f].at[sl])

        @pl.when(c == 0)
        def _():
            pltpu.sync_copy(bufO, aS[f].at[sl])

        @pl.when(c == 1)
        def _():
            pltpu.sync_copy(bufA, aS[f].at[sl])

    plsc.subcore_barrier()

    def _idx(off, ir, ic):
        pltpu.sync_copy(row_hbm.at[pl.ds(off, CH)], ir)
        pltpu.sync_copy(col_hbm.at[pl.ds(off, CH)], ic)

    def _gath(ir, g0, g1, g2):
        pltpu.async_copy(zS0.at[ir], g0, sem)
        pltpu.async_copy(zS1.at[ir], g1, sem)
        pltpu.async_copy(zS2.at[ir], g2, sem)

    def _gwait(ir, g0, g1, g2):
        pltpu.make_async_copy(zS0.at[ir], g0, sem).wait()
        pltpu.make_async_copy(zS1.at[ir], g1, sem).wait()
        pltpu.make_async_copy(zS2.at[ir], g2, sem).wait()

    def _scat(ic, g0, g1, g2):
        pltpu.sync_copy(g0, aS0.at[ic], add=True)
        pltpu.sync_copy(g1, aS1.at[ic], add=True)
        pltpu.sync_copy(g2, aS2.at[ic], add=True)

    base = wid * _cEPW
    _idx(base, rowv0, colv0)
    _gath(rowv0, gva0, gva1, gva2)

    def _pair(k, _):
        offa = base + k * (2 * _cCH)
        offb = offa + _cCH
        offc = offa + 2 * _cCH
        _idx(offb, rowv1, colv1)
        _gwait(rowv0, gva0, gva1, gva2)
        _gath(rowv1, gvb0, gvb1, gvb2)
        _scat(colv0, gva0, gva1, gva2)

        @pl.when(k < NCHUNK // 2 - 1)
        def _():
            _idx(offc, rowv0, colv0)
        _gwait(rowv1, gvb0, gvb1, gvb2)

        @pl.when(k < NCHUNK // 2 - 1)
        def _():
            _gath(rowv0, gva0, gva1, gva2)
        _scat(colv1, gvb0, gvb1, gvb2)
        return np.int32(0)
    lax.fori_loop(np.int32(0), np.int32(NCHUNK // 2), _pair, np.int32(0))

    plsc.subcore_barrier()

    for f in range(3):
        @pl.when(c == 0)
        def _():
            pltpu.sync_copy(aS[f].at[sl], oAs[f].at[sl])

        @pl.when(c == 1)
        def _():
            pltpu.sync_copy(aS[f].at[sl], oBs[f].at[sl])


@functools.cache
def _hop_kernel():
  return pl.kernel(
    _hop_body,
    out_type=tuple(
        jax.ShapeDtypeStruct((NPAD,), _f32) for _ in range(6)),
    mesh=_mesh(),
    scratch_types=(
        pltpu.VMEM((CH,), _i32),
        pltpu.VMEM((CH,), _i32),
        pltpu.VMEM((CH,), _i32),
        pltpu.VMEM((CH,), _i32),
        pltpu.VMEM((CH,), _f32),
        pltpu.VMEM((CH,), _f32),
        pltpu.VMEM((CH,), _f32),
        pltpu.VMEM((CH,), _f32),
        pltpu.VMEM((CH,), _f32),
        pltpu.VMEM((CH,), _f32),
        pltpu.VMEM((SL,), _f32),
        pltpu.VMEM((SL,), _f32),
        pltpu.VMEM((SL,), _f32),
        pltpu.VMEM((SL,), _f32),
        pltpu.VMEM_SHARED((NPAD,), _f32),
        pltpu.VMEM_SHARED((NPAD,), _f32),
        pltpu.VMEM_SHARED((NPAD,), _f32),
        pltpu.VMEM_SHARED((NPAD,), _f32),
        pltpu.VMEM_SHARED((NPAD,), _f32),
        pltpu.VMEM_SHARED((NPAD,), _f32),
        pltpu.SemaphoreType.DMA,
    ),
  )


TB = 2048
NTB = NPAD // TB


def _prep_body(degA, degB, xT, w_o, dinv_o, zinit_o):
    deg = degA[...] + degB[...] + 1.0
    sq = jnp.sqrt(deg)
    dinv = 1.0 / sq
    w_o[...] = dinv * dinv
    dinv_o[...] = dinv
    zinit_o[...] = xT[...] * sq


_prep_kernel = pl.pallas_call(
    _prep_body,
    grid=(NTB,),
    in_specs=[
        pl.BlockSpec((1, TB), lambda t: (0, t)),
        pl.BlockSpec((1, TB), lambda t: (0, t)),
        pl.BlockSpec((3, TB), lambda t: (0, t)),
    ],
    out_specs=[
        pl.BlockSpec((1, TB), lambda t: (0, t)),
        pl.BlockSpec((1, TB), lambda t: (0, t)),
        pl.BlockSpec((3, TB), lambda t: (0, t)),
    ],
    out_shape=[
        jax.ShapeDtypeStruct((1, NPAD), _f32),
        jax.ShapeDtypeStruct((1, NPAD), _f32),
        jax.ShapeDtypeStruct((3, NPAD), _f32),
    ],
)


def _stats_body(aA, aB, dinv, wsgcT, bn1g, bn1b, p_o, wp_o, acc):
    t = pl.program_id(0)

    @pl.when(t == 0)
    def _():
        for k in range(10):
            acc[k] = 0.0

    p = dinv[...] * (aA[...] + aB[...])
    p_o[...] = p
    idx = 0
    for f in range(3):
        acc[idx] += jnp.sum(p[f:f + 1, :])
        idx += 1
    for f in range(3):
        for g in range(f, 3):
            acc[idx] += jnp.sum(p[f:f + 1, :] * p[g:g + 1, :])
            idx += 1

    @pl.when(t == NTB - 1)
    def _():
        n = jnp.float32(N)
        mu = [acc[f] / n for f in range(3)]
        raw = {}
        idx2 = 3
        for f in range(3):
            for g in range(f, 3):
                raw[(f, g)] = acc[idx2] / n
                idx2 += 1
        WT = wsgcT[...]
        v = jnp.zeros((H1, 1), _f32)
        wmu = jnp.zeros((H1, 1), _f32)
        for f in range(3):
            wmu = wmu + WT[:, f:f + 1] * mu[f]
            for g in range(3):
                cov = raw[(min(f, g), max(f, g))] - mu[f] * mu[g]
                v = v + (WT[:, f:f + 1] * WT[:, g:g + 1]) * cov
        gam = bn1g[...] * lax.rsqrt(v + EPS)
        bprime = bn1b[...] - gam * wmu
        wp_o[...] = jnp.concatenate(
            [WT * gam, bprime, jnp.zeros((H1, 4), _f32)], axis=1)


_stats_kernel = pl.pallas_call(
    _stats_body,
    grid=(NTB,),
    in_specs=[
        pl.BlockSpec((3, TB), lambda t: (0, t)),
        pl.BlockSpec((3, TB), lambda t: (0, t)),
        pl.BlockSpec((1, TB), lambda t: (0, t)),
        pl.BlockSpec((H1, 3), lambda t: (0, 0)),
        pl.BlockSpec((H1, 1), lambda t: (0, 0)),
        pl.BlockSpec((H1, 1), lambda t: (0, 0)),
    ],
    out_specs=[
        pl.BlockSpec((3, TB), lambda t: (0, t)),
        pl.BlockSpec((H1, 8), lambda t: (0, 0)),
    ],
    out_shape=[
        jax.ShapeDtypeStruct((3, NPAD), _f32),
        jax.ShapeDtypeStruct((H1, 8), _f32),
    ],
    scratch_shapes=[pltpu.SMEM((16,), _f32)],
)


TE = 1024
NTE = NPAD // TE
NEG = float("-inf")


def _head_body(p, bt, wp, w1T, b1c, bn2g, bn2b, w2T, b2c, bn3g, bn3b,
               w3T, b3c, cntA, cntB, o_ref, pool):
    t = pl.program_id(0)

    @pl.when(t == 0)
    def _():
        pool[...] = jnp.full((H1, B), NEG, _f32)

    @pl.when(t < NTE)
    def _():
        wpv = wp[...]
        pv = p[...]
        zT = (wpv[:, 0:1] * pv[0:1, :] + wpv[:, 1:2] * pv[1:2, :]
              + wpv[:, 2:3] * pv[2:3, :])
        b = bt[...]
        lo = jnp.min(b)
        hi = jnp.minimum(jnp.max(b), B - 1)
        lane = lax.broadcasted_iota(_i32, (H1, B), 1)

        def _seg(sg, _):
            m = jnp.max(jnp.where(b == sg, zT, NEG), axis=1,
                        keepdims=True)
            cur = pool[...]
            pool[...] = jnp.where(lane == sg, jnp.maximum(cur, m), cur)
            return np.int32(0)
        lax.fori_loop(lo, hi + 1, _seg, np.int32(0))

    @pl.when(t == NTE)
    def _():
        cnt = (cntA[...] + cntB[...])[:, :B]
        pooled = pool[...] + wp[...][:, 3:4]
        pooled = jnp.where(cnt > 0.0, jnp.maximum(pooled, 0.0), NEG)

        h = jnp.dot(w1T[...], pooled, precision=lax.Precision.HIGHEST,
                    preferred_element_type=_f32) + b1c[...]
        m = jnp.mean(h, axis=1, keepdims=True)
        v = jnp.mean((h - m) ** 2, axis=1, keepdims=True)
        h = (h - m) * lax.rsqrt(v + EPS) * bn2g[...] + bn2b[...]
        h = jnp.maximum(h, 0.0)

        h = jnp.dot(w2T[...], h, precision=lax.Precision.HIGHEST,
                    preferred_element_type=_f32) + b2c[...]
        m = jnp.mean(h, axis=1, keepdims=True)
        v = jnp.mean((h - m) ** 2, axis=1, keepdims=True)
        h = (h - m) * lax.rsqrt(v + EPS) * bn3g[...] + bn3b[...]
        h = jnp.maximum(h, 0.0)

        h = jnp.dot(w3T[...], h, precision=lax.Precision.HIGHEST,
                    preferred_element_type=_f32) + b3c[...]
        mx = jnp.max(h, axis=0, keepdims=True)
        lse = jnp.log(jnp.sum(jnp.exp(h - mx), axis=0, keepdims=True)) + mx
        o_ref[...] = h - lse


_head_kernel = pl.pallas_call(
    _head_body,
    grid=(NTE + 1,),
    in_specs=[
        pl.BlockSpec((3, TE), lambda t: (0, jnp.minimum(t, NTE - 1))),
        pl.BlockSpec((1, TE), lambda t: (0, jnp.minimum(t, NTE - 1))),
        pl.BlockSpec((H1, 8), lambda t: (0, 0)),
        pl.BlockSpec((H2, H1), lambda t: (0, 0)),
        pl.BlockSpec((H2, 1), lambda t: (0, 0)),
        pl.BlockSpec((H2, 1), lambda t: (0, 0)),
        pl.BlockSpec((H2, 1), lambda t: (0, 0)),
        pl.BlockSpec((H3, H2), lambda t: (0, 0)),
        pl.BlockSpec((H3, 1), lambda t: (0, 0)),
        pl.BlockSpec((H3, 1), lambda t: (0, 0)),
        pl.BlockSpec((H3, 1), lambda t: (0, 0)),
        pl.BlockSpec((CLS, H3), lambda t: (0, 0)),
        pl.BlockSpec((CLS, 1), lambda t: (0, 0)),
        pl.BlockSpec((1, 128), lambda t: (0, 0)),
        pl.BlockSpec((1, 128), lambda t: (0, 0)),
    ],
    out_specs=pl.BlockSpec((CLS, B), lambda t: (0, 0)),
    out_shape=jax.ShapeDtypeStruct((CLS, B), _f32),
    scratch_shapes=[pltpu.VMEM((H1, B), _f32)],
)


def kernel(x, edge_index, batch, W_sgc, b_sgc, bn1_g, bn1_b, W1, b1,
           bn2_g, bn2_b, W2, b2, bn3_g, bn3_b, W3, b3):
    with jax.enable_x64(False):
        out = _kernel_impl(x, edge_index, batch, W_sgc, b_sgc, bn1_g,
                           bn1_b, W1, b1, bn2_g, bn2_b, W2, b2, bn3_g,
                           bn3_b, W3, b3)
    return out.astype(jnp.float64)


def _kernel_impl(x, edge_index, batch, W_sgc, b_sgc, bn1_g, bn1_b, W1, b1,
                 bn2_g, bn2_b, W2, b2, bn3_g, bn3_b, W3, b3):
    ei = edge_index.astype(_i32)
    npad = NPAD - N
    epad = EPAD - E
    pad_idx = (N + (jnp.arange(epad, dtype=_i32) % npad)).astype(_i32)
    row = jnp.concatenate([ei[0], pad_idx])
    col = jnp.concatenate([ei[1], pad_idx])

    b32 = batch.astype(_i32)
    batch_sc = jnp.concatenate([b32, jnp.full((npad,), 64, _i32)])
    batch_tc = jnp.concatenate(
        [b32, jnp.full((npad,), 9999, _i32)]).reshape(1, NPAD)

    xT = jnp.pad(x.astype(_f32).T, ((0, 0), (0, npad)))

    degA, degB, cntA, cntB = _deg_kernel()(col, batch_sc)
    w1d, dinv1d, zinit = _prep_kernel(
        degA.reshape(1, NPAD), degB.reshape(1, NPAD), xT)
    w_flat = w1d.reshape(NPAD)

    z1 = jnp.zeros((NPAD,), _f32)
    st = [zinit[0], zinit[1], zinit[2], z1, z1, z1]
    for _ in range(K):
        st = list(_hop_kernel()(row, col, w_flat, *st))
    aA = jnp.stack(st[0:3])
    aB = jnp.stack(st[3:6])

    wsgcT = W_sgc.astype(_f32).T
    bn1g2 = (bn1_g.astype(_f32) * jnp.ones((1,), _f32)).reshape(H1, 1)
    bn1b2 = bn1_b.astype(_f32).reshape(H1, 1)
    p3, wp = _stats_kernel(aA, aB, dinv1d, wsgcT, bn1g2, bn1b2)

    oT = _head_kernel(
        p3, batch_tc, wp,
        W1.astype(_f32).T, b1.astype(_f32).reshape(H2, 1),
        bn2_g.astype(_f32).reshape(H2, 1), bn2_b.astype(_f32).reshape(H2, 1),
        W2.astype(_f32).T, b2.astype(_f32).reshape(H3, 1),
        bn3_g.astype(_f32).reshape(H3, 1), bn3_b.astype(_f32).reshape(H3, 1),
        W3.astype(_f32).T, b3.astype(_f32).reshape(CLS, 1),
        cntA.reshape(1, 128), cntB.reshape(1, 128),
    )
    return oT.T


# speedup vs baseline: 1.0288x; 1.0288x over previous
import functools

import jax
import jax.numpy as jnp
import numpy as np
from jax import lax
from jax.experimental import pallas as pl
from jax.experimental.pallas import tpu as pltpu
from jax.experimental.pallas import tpu_sc as plsc

N = 100000
E = 6400000
B = 64
K = 5
D_IN = 3
H1 = 1024
H2 = 512
H3 = 256
CLS = 40
EPS = 1e-5

NC = 2
NS = 16
NW = NC * NS
NPAD = 100352
SL = NPAD // NS
SLW = NPAD // NW
CH = 8192
NCHUNK = 25
EPW = CH * NCHUNK
EPAD = EPW * NW

@functools.cache
def _mesh():
    return plsc.VectorSubcoreMesh(
        core_axis_name="c", subcore_axis_name="s",
        num_cores=NC, num_subcores=NS)


_f32 = jnp.float32
_i32 = jnp.int32


def _I(v):
    return jnp.int32(v)


_c16 = 16
_cNS = NS
_cSL = SL
_cSLW = SLW
_cCH = CH
_cEPW = EPW


def _deg_body(col_hbm, batch_hbm, degA, degB, cntA, cntB,
              colv, batv, onesv, zbuf, degS, cntS, sem):
    c = lax.axis_index("c")
    s = lax.axis_index("s")
    wid = c * _cNS + s
    sl = pl.ds(s * _cSL, SL)

    def _fz(i, _):
        zbuf[pl.ds(i * _c16, 16)] = jnp.zeros((16,), _f32)
        return np.int32(0)
    lax.fori_loop(np.int32(0), np.int32(SL // 16), _fz, np.int32(0))

    def _fo(i, _):
        onesv[pl.ds(i * _c16, 16)] = jnp.ones((16,), _f32)
        return np.int32(0)
    lax.fori_loop(np.int32(0), np.int32(CH // 16), _fo, np.int32(0))

    pltpu.sync_copy(zbuf.at[pl.ds(0, SL)], degS.at[sl])

    @pl.when(s == 0)
    def _():
        pltpu.sync_copy(zbuf.at[pl.ds(0, 128)], cntS)

    plsc.subcore_barrier()

    def _chunk(k, _):
        off = wid * _cEPW + k * _cCH
        pltpu.sync_copy(col_hbm.at[pl.ds(off, CH)], colv)
        pltpu.sync_copy(onesv, degS.at[colv], add=True)
        return np.int32(0)
    lax.fori_loop(np.int32(0), np.int32(NCHUNK), _chunk, np.int32(0))

    pltpu.sync_copy(batch_hbm.at[pl.ds(wid * _cSLW, SLW)], batv)
    pltpu.sync_copy(onesv.at[pl.ds(0, SLW)], cntS.at[batv], add=True)

    plsc.subcore_barrier()

    @pl.when(c == 0)
    def _():
        pltpu.sync_copy(degS.at[sl], degA.at[sl])

        @pl.when(s == 0)
        def _():
            pltpu.sync_copy(cntS, cntA)

    @pl.when(c == 1)
    def _():
        pltpu.sync_copy(degS.at[sl], degB.at[sl])

        @pl.when(s == 0)
        def _():
            pltpu.sync_copy(cntS, cntB)


@functools.cache
def _deg_kernel():
  return pl.kernel(
    _deg_body,
    out_type=(
        jax.ShapeDtypeStruct((NPAD,), _f32),
        jax.ShapeDtypeStruct((NPAD,), _f32),
        jax.ShapeDtypeStruct((128,), _f32),
        jax.ShapeDtypeStruct((128,), _f32),
    ),
    mesh=_mesh(),
    scratch_types=(
        pltpu.VMEM((CH,), _i32),
        pltpu.VMEM((SLW,), _i32),
        pltpu.VMEM((CH,), _f32),
        pltpu.VMEM((SL,), _f32),
        pltpu.VMEM_SHARED((NPAD,), _f32),
        pltpu.VMEM_SHARED((128,), _f32),
        pltpu.SemaphoreType.DMA,
    ),
  )


def _hop_body(row_hbm, col_hbm, w_hbm,
              aA0, aA1, aA2, aB0, aB1, aB2,
              oA0, oA1, oA2, oB0, oB1, oB2,
              rowv0, colv0, rowv1, colv1,
              gva0, gva1, gva2, gvb0, gvb1, gvb2,
              bufA, bufB, bufW, bufO,
              zS0, zS1, zS2, aS0, aS1, aS2, sem):
    c = lax.axis_index("c")
    s = lax.axis_index("s")
    wid = c * _cNS + s
    sl = pl.ds(s * _cSL, SL)
    aAs = (aA0, aA1, aA2)
    aBs = (aB0, aB1, aB2)
    oAs = (oA0, oA1, oA2)
    oBs = (oB0, oB1, oB2)
    zS = (zS0, zS1, zS2)
    aS = (aS0, aS1, aS2)

    pltpu.sync_copy(w_hbm.at[sl], bufW)
    for f in range(3):
        pltpu.sync_copy(aAs[f].at[sl], bufA)
        pltpu.sync_copy(aBs[f].at[sl], bufB)

        def _cb(i, _):
            ix = pl.ds(i * _c16, 16)
            bufO[ix] = (bufA[ix] + bufB[ix]) * bufW[ix]
            bufA[ix] = jnp.zeros((16,), _f32)
            return np.int32(0)
        lax.fori_loop(np.int32(0), np.int32(SL // 16), _cb, np.int32(0))

        pltpu.sync_copy(bufO, zS[f].at[sl])

        @pl.when(c == 0)
        def _():
            pltpu.sync_copy(bufO, aS[f].at[sl])

        @pl.when(c == 1)
        def _():
            pltpu.sync_copy(bufA, aS[f].at[sl])

    plsc.subcore_barrier()

    def _idx(off, ir, ic):
        pltpu.sync_copy(row_hbm.at[pl.ds(off, CH)], ir)
        pltpu.sync_copy(col_hbm.at[pl.ds(off, CH)], ic)

    def _gath(ir, g0, g1, g2):
        pltpu.async_copy(zS0.at[ir], g0, sem)
        pltpu.async_copy(zS1.at[ir], g1, sem)
        pltpu.async_copy(zS2.at[ir], g2, sem)

    def _gwait(ir, g0, g1, g2):
        pltpu.make_async_copy(zS0.at[ir], g0, sem).wait()
        pltpu.make_async_copy(zS1.at[ir], g1, sem).wait()
        pltpu.make_async_copy(zS2.at[ir], g2, sem).wait()

    def _scat(ic, g0, g1, g2):
        pltpu.sync_copy(g0, aS0.at[ic], add=True)
        pltpu.sync_copy(g1, aS1.at[ic], add=True)
        pltpu.sync_copy(g2, aS2.at[ic], add=True)

    base = wid * _cEPW

    def _chunk(k, _):
        off = base + k * _cCH
        _idx(off, rowv0, colv0)
        _gath(rowv0, gva0, gva1, gva2)
        _gwait(rowv0, gva0, gva1, gva2)
        _scat(colv0, gva0, gva1, gva2)
        return np.int32(0)
    lax.fori_loop(np.int32(0), np.int32(NCHUNK), _chunk, np.int32(0))

    plsc.subcore_barrier()

    for f in range(3):
        @pl.when(c == 0)
        def _():
            pltpu.sync_copy(aS[f].at[sl], oAs[f].at[sl])

        @pl.when(c == 1)
        def _():
            pltpu.sync_copy(aS[f].at[sl], oBs[f].at[sl])


@functools.cache
def _hop_kernel():
  return pl.kernel(
    _hop_body,
    out_type=tuple(
        jax.ShapeDtypeStruct((NPAD,), _f32) for _ in range(6)),
    mesh=_mesh(),
    scratch_types=(
        pltpu.VMEM((CH,), _i32),
        pltpu.VMEM((CH,), _i32),
        pltpu.VMEM((CH,), _i32),
        pltpu.VMEM((CH,), _i32),
        pltpu.VMEM((CH,), _f32),
        pltpu.VMEM((CH,), _f32),
        pltpu.VMEM((CH,), _f32),
        pltpu.VMEM((CH,), _f32),
        pltpu.VMEM((CH,), _f32),
        pltpu.VMEM((CH,), _f32),
        pltpu.VMEM((SL,), _f32),
        pltpu.VMEM((SL,), _f32),
        pltpu.VMEM((SL,), _f32),
        pltpu.VMEM((SL,), _f32),
        pltpu.VMEM_SHARED((NPAD,), _f32),
        pltpu.VMEM_SHARED((NPAD,), _f32),
        pltpu.VMEM_SHARED((NPAD,), _f32),
        pltpu.VMEM_SHARED((NPAD,), _f32),
        pltpu.VMEM_SHARED((NPAD,), _f32),
        pltpu.VMEM_SHARED((NPAD,), _f32),
        pltpu.SemaphoreType.DMA,
    ),
  )


TB = 2048
NTB = NPAD // TB


def _prep_body(degA, degB, xT, w_o, dinv_o, zinit_o):
    deg = degA[...] + degB[...] + 1.0
    sq = jnp.sqrt(deg)
    dinv = 1.0 / sq
    w_o[...] = dinv * dinv
    dinv_o[...] = dinv
    zinit_o[...] = xT[...] * sq


_prep_kernel = pl.pallas_call(
    _prep_body,
    grid=(NTB,),
    in_specs=[
        pl.BlockSpec((1, TB), lambda t: (0, t)),
        pl.BlockSpec((1, TB), lambda t: (0, t)),
        pl.BlockSpec((3, TB), lambda t: (0, t)),
    ],
    out_specs=[
        pl.BlockSpec((1, TB), lambda t: (0, t)),
        pl.BlockSpec((1, TB), lambda t: (0, t)),
        pl.BlockSpec((3, TB), lambda t: (0, t)),
    ],
    out_shape=[
        jax.ShapeDtypeStruct((1, NPAD), _f32),
        jax.ShapeDtypeStruct((1, NPAD), _f32),
        jax.ShapeDtypeStruct((3, NPAD), _f32),
    ],
)


def _stats_body(aA, aB, dinv, wsgcT, bn1g, bn1b, p_o, wp_o, acc):
    t = pl.program_id(0)

    @pl.when(t == 0)
    def _():
        for k in range(10):
            acc[k] = 0.0

    p = dinv[...] * (aA[...] + aB[...])
    p_o[...] = p
    idx = 0
    for f in range(3):
        acc[idx] += jnp.sum(p[f:f + 1, :])
        idx += 1
    for f in range(3):
        for g in range(f, 3):
            acc[idx] += jnp.sum(p[f:f + 1, :] * p[g:g + 1, :])
            idx += 1

    @pl.when(t == NTB - 1)
    def _():
        n = jnp.float32(N)
        mu = [acc[f] / n for f in range(3)]
        raw = {}
        idx2 = 3
        for f in range(3):
            for g in range(f, 3):
                raw[(f, g)] = acc[idx2] / n
                idx2 += 1
        WT = wsgcT[...]
        v = jnp.zeros((H1, 1), _f32)
        wmu = jnp.zeros((H1, 1), _f32)
        for f in range(3):
            wmu = wmu + WT[:, f:f + 1] * mu[f]
            for g in range(3):
                cov = raw[(min(f, g), max(f, g))] - mu[f] * mu[g]
                v = v + (WT[:, f:f + 1] * WT[:, g:g + 1]) * cov
        gam = bn1g[...] * lax.rsqrt(v + EPS)
        bprime = bn1b[...] - gam * wmu
        wp_o[...] = jnp.concatenate(
            [WT * gam, bprime, jnp.zeros((H1, 4), _f32)], axis=1)


_stats_kernel = pl.pallas_call(
    _stats_body,
    grid=(NTB,),
    in_specs=[
        pl.BlockSpec((3, TB), lambda t: (0, t)),
        pl.BlockSpec((3, TB), lambda t: (0, t)),
        pl.BlockSpec((1, TB), lambda t: (0, t)),
        pl.BlockSpec((H1, 3), lambda t: (0, 0)),
        pl.BlockSpec((H1, 1), lambda t: (0, 0)),
        pl.BlockSpec((H1, 1), lambda t: (0, 0)),
    ],
    out_specs=[
        pl.BlockSpec((3, TB), lambda t: (0, t)),
        pl.BlockSpec((H1, 8), lambda t: (0, 0)),
    ],
    out_shape=[
        jax.ShapeDtypeStruct((3, NPAD), _f32),
        jax.ShapeDtypeStruct((H1, 8), _f32),
    ],
    scratch_shapes=[pltpu.SMEM((16,), _f32)],
)


TE = 1024
NTE = NPAD // TE
NEG = float("-inf")


def _head_body(p, bt, wp, w1T, b1c, bn2g, bn2b, w2T, b2c, bn3g, bn3b,
               w3T, b3c, cntA, cntB, o_ref, pool):
    t = pl.program_id(0)

    @pl.when(t == 0)
    def _():
        pool[...] = jnp.full((H1, B), NEG, _f32)

    @pl.when(t < NTE)
    def _():
        wpv = wp[...]
        pv = p[...]
        zT = (wpv[:, 0:1] * pv[0:1, :] + wpv[:, 1:2] * pv[1:2, :]
              + wpv[:, 2:3] * pv[2:3, :])
        b = bt[...]
        lo = jnp.min(b)
        hi = jnp.minimum(jnp.max(b), B - 1)
        lane = lax.broadcasted_iota(_i32, (H1, B), 1)

        def _seg(sg, _):
            m = jnp.max(jnp.where(b == sg, zT, NEG), axis=1,
                        keepdims=True)
            cur = pool[...]
            pool[...] = jnp.where(lane == sg, jnp.maximum(cur, m), cur)
            return np.int32(0)
        lax.fori_loop(lo, hi + 1, _seg, np.int32(0))

    @pl.when(t == NTE)
    def _():
        cnt = (cntA[...] + cntB[...])[:, :B]
        pooled = pool[...] + wp[...][:, 3:4]
        pooled = jnp.where(cnt > 0.0, jnp.maximum(pooled, 0.0), NEG)

        h = jnp.dot(w1T[...], pooled, precision=lax.Precision.HIGHEST,
                    preferred_element_type=_f32) + b1c[...]
        m = jnp.mean(h, axis=1, keepdims=True)
        v = jnp.mean((h - m) ** 2, axis=1, keepdims=True)
        h = (h - m) * lax.rsqrt(v + EPS) * bn2g[...] + bn2b[...]
        h = jnp.maximum(h, 0.0)

        h = jnp.dot(w2T[...], h, precision=lax.Precision.HIGHEST,
                    preferred_element_type=_f32) + b2c[...]
        m = jnp.mean(h, axis=1, keepdims=True)
        v = jnp.mean((h - m) ** 2, axis=1, keepdims=True)
        h = (h - m) * lax.rsqrt(v + EPS) * bn3g[...] + bn3b[...]
        h = jnp.maximum(h, 0.0)

        h = jnp.dot(w3T[...], h, precision=lax.Precision.HIGHEST,
                    preferred_element_type=_f32) + b3c[...]
        mx = jnp.max(h, axis=0, keepdims=True)
        lse = jnp.log(jnp.sum(jnp.exp(h - mx), axis=0, keepdims=True)) + mx
        o_ref[...] = h - lse


_head_kernel = pl.pallas_call(
    _head_body,
    grid=(NTE + 1,),
    in_specs=[
        pl.BlockSpec((3, TE), lambda t: (0, jnp.minimum(t, NTE - 1))),
        pl.BlockSpec((1, TE), lambda t: (0, jnp.minimum(t, NTE - 1))),
        pl.BlockSpec((H1, 8), lambda t: (0, 0)),
        pl.BlockSpec((H2, H1), lambda t: (0, 0)),
        pl.BlockSpec((H2, 1), lambda t: (0, 0)),
        pl.BlockSpec((H2, 1), lambda t: (0, 0)),
        pl.BlockSpec((H2, 1), lambda t: (0, 0)),
        pl.BlockSpec((H3, H2), lambda t: (0, 0)),
        pl.BlockSpec((H3, 1), lambda t: (0, 0)),
        pl.BlockSpec((H3, 1), lambda t: (0, 0)),
        pl.BlockSpec((H3, 1), lambda t: (0, 0)),
        pl.BlockSpec((CLS, H3), lambda t: (0, 0)),
        pl.BlockSpec((CLS, 1), lambda t: (0, 0)),
        pl.BlockSpec((1, 128), lambda t: (0, 0)),
        pl.BlockSpec((1, 128), lambda t: (0, 0)),
    ],
    out_specs=pl.BlockSpec((CLS, B), lambda t: (0, 0)),
    out_shape=jax.ShapeDtypeStruct((CLS, B), _f32),
    scratch_shapes=[pltpu.VMEM((H1, B), _f32)],
)


def kernel(x, edge_index, batch, W_sgc, b_sgc, bn1_g, bn1_b, W1, b1,
           bn2_g, bn2_b, W2, b2, bn3_g, bn3_b, W3, b3):
    with jax.enable_x64(False):
        out = _kernel_impl(x, edge_index, batch, W_sgc, b_sgc, bn1_g,
                           bn1_b, W1, b1, bn2_g, bn2_b, W2, b2, bn3_g,
                           bn3_b, W3, b3)
    return out.astype(jnp.float64)


def _kernel_impl(x, edge_index, batch, W_sgc, b_sgc, bn1_g, bn1_b, W1, b1,
                 bn2_g, bn2_b, W2, b2, bn3_g, bn3_b, W3, b3):
    ei = edge_index.astype(_i32)
    npad = NPAD - N
    epad = EPAD - E
    pad_idx = (N + (jnp.arange(epad, dtype=_i32) % npad)).astype(_i32)
    row = jnp.concatenate([ei[0], pad_idx])
    col = jnp.concatenate([ei[1], pad_idx])

    b32 = batch.astype(_i32)
    batch_sc = jnp.concatenate([b32, jnp.full((npad,), 64, _i32)])
    batch_tc = jnp.concatenate(
        [b32, jnp.full((npad,), 9999, _i32)]).reshape(1, NPAD)

    xT = jnp.pad(x.astype(_f32).T, ((0, 0), (0, npad)))

    degA, degB, cntA, cntB = _deg_kernel()(col, batch_sc)
    w1d, dinv1d, zinit = _prep_kernel(
        degA.reshape(1, NPAD), degB.reshape(1, NPAD), xT)
    w_flat = w1d.reshape(NPAD)

    z1 = jnp.zeros((NPAD,), _f32)
    st = [zinit[0], zinit[1], zinit[2], z1, z1, z1]
    for _ in range(K):
        st = list(_hop_kernel()(row, col, w_flat, *st))
    aA = jnp.stack(st[0:3])
    aB = jnp.stack(st[3:6])

    wsgcT = W_sgc.astype(_f32).T
    bn1g2 = (bn1_g.astype(_f32) * jnp.ones((1,), _f32)).reshape(H1, 1)
    bn1b2 = bn1_b.astype(_f32).reshape(H1, 1)
    p3, wp = _stats_kernel(aA, aB, dinv1d, wsgcT, bn1g2, bn1b2)

    oT = _head_kernel(
        p3, batch_tc, wp,
        W1.astype(_f32).T, b1.astype(_f32).reshape(H2, 1),
        bn2_g.astype(_f32).reshape(H2, 1), bn2_b.astype(_f32).reshape(H2, 1),
        W2.astype(_f32).T, b2.astype(_f32).reshape(H3, 1),
        bn3_g.astype(_f32).reshape(H3, 1), bn3_b.astype(_f32).reshape(H3, 1),
        W3.astype(_f32).T, b3.astype(_f32).reshape(CLS, 1),
        cntA.reshape(1, 128), cntB.reshape(1, 128),
    )
    return oT.T


# speedup vs baseline: 1.0637x; 1.0339x over previous
import functools

import jax
import jax.numpy as jnp
import numpy as np
from jax import lax
from jax.experimental import pallas as pl
from jax.experimental.pallas import tpu as pltpu
from jax.experimental.pallas import tpu_sc as plsc

N = 100000
E = 6400000
B = 64
K = 5
D_IN = 3
H1 = 1024
H2 = 512
H3 = 256
CLS = 40
EPS = 1e-5

NC = 2
NS = 16
NW = NC * NS
NPAD = 100352
SL = NPAD // NS
SLW = NPAD // NW
CH = 8192
NCHUNK = 25
EPW = E // NW
NFULL = EPW // CH
CTAIL = EPW - NFULL * CH

@functools.cache
def _mesh():
    return plsc.VectorSubcoreMesh(
        core_axis_name="c", subcore_axis_name="s",
        num_cores=NC, num_subcores=NS)


_f32 = jnp.float32
_i32 = jnp.int32


def _I(v):
    return jnp.int32(v)


_c16 = 16
_cNS = NS
_cSL = SL
_cSLW = SLW
_cCH = CH
_cEPW = EPW


def _deg_body(col_hbm, batch_hbm, degA, degB, cntA, cntB,
              colv, colt, batv, onesv, zbuf, degS, cntS, sem):
    c = lax.axis_index("c")
    s = lax.axis_index("s")
    wid = c * _cNS + s
    sl = pl.ds(s * _cSL, SL)

    def _fz(i, _):
        zbuf[pl.ds(i * _c16, 16)] = jnp.zeros((16,), _f32)
        return np.int32(0)
    lax.fori_loop(np.int32(0), np.int32(SL // 16), _fz, np.int32(0))

    def _fo(i, _):
        onesv[pl.ds(i * _c16, 16)] = jnp.ones((16,), _f32)
        return np.int32(0)
    lax.fori_loop(np.int32(0), np.int32(CH // 16), _fo, np.int32(0))

    pltpu.sync_copy(zbuf.at[pl.ds(0, SL)], degS.at[sl])

    @pl.when(s == 0)
    def _():
        pltpu.sync_copy(zbuf.at[pl.ds(0, 128)], cntS)

    plsc.subcore_barrier()

    def _chunk(k, _):
        off = wid * _cEPW + k * _cCH
        pltpu.sync_copy(col_hbm.at[pl.ds(off, CH)], colv)
        pltpu.sync_copy(onesv, degS.at[colv], add=True)
        return np.int32(0)
    lax.fori_loop(np.int32(0), np.int32(NFULL), _chunk, np.int32(0))
    toff = wid * _cEPW + NFULL * _cCH
    pltpu.sync_copy(col_hbm.at[pl.ds(toff, CTAIL)], colt)
    pltpu.sync_copy(onesv.at[pl.ds(0, CTAIL)], degS.at[colt], add=True)

    pltpu.sync_copy(batch_hbm.at[pl.ds(wid * _cSLW, SLW)], batv)
    pltpu.sync_copy(onesv.at[pl.ds(0, SLW)], cntS.at[batv], add=True)

    plsc.subcore_barrier()

    @pl.when(c == 0)
    def _():
        pltpu.sync_copy(degS.at[sl], degA.at[sl])

        @pl.when(s == 0)
        def _():
            pltpu.sync_copy(cntS, cntA)

    @pl.when(c == 1)
    def _():
        pltpu.sync_copy(degS.at[sl], degB.at[sl])

        @pl.when(s == 0)
        def _():
            pltpu.sync_copy(cntS, cntB)


@functools.cache
def _deg_kernel():
  return pl.kernel(
    _deg_body,
    out_type=(
        jax.ShapeDtypeStruct((NPAD,), _f32),
        jax.ShapeDtypeStruct((NPAD,), _f32),
        jax.ShapeDtypeStruct((128,), _f32),
        jax.ShapeDtypeStruct((128,), _f32),
    ),
    mesh=_mesh(),
    scratch_types=(
        pltpu.VMEM((CH,), _i32),
        pltpu.VMEM((CTAIL,), _i32),
        pltpu.VMEM((SLW,), _i32),
        pltpu.VMEM((CH,), _f32),
        pltpu.VMEM((SL,), _f32),
        pltpu.VMEM_SHARED((NPAD,), _f32),
        pltpu.VMEM_SHARED((128,), _f32),
        pltpu.SemaphoreType.DMA,
    ),
  )


def _hop_body(row_hbm, col_hbm, w_hbm,
              aA0, aA1, aA2, aB0, aB1, aB2,
              oA0, oA1, oA2, oB0, oB1, oB2,
              rowv0, colv0, rowt, colt,
              gva0, gva1, gva2, gt0, gt1, gt2,
              bufA, bufB, bufW, bufO,
              zS0, zS1, zS2, aS0, aS1, aS2, sem):
    c = lax.axis_index("c")
    s = lax.axis_index("s")
    wid = c * _cNS + s
    sl = pl.ds(s * _cSL, SL)
    aAs = (aA0, aA1, aA2)
    aBs = (aB0, aB1, aB2)
    oAs = (oA0, oA1, oA2)
    oBs = (oB0, oB1, oB2)
    zS = (zS0, zS1, zS2)
    aS = (aS0, aS1, aS2)

    pltpu.sync_copy(w_hbm.at[sl], bufW)
    for f in range(3):
        pltpu.sync_copy(aAs[f].at[sl], bufA)
        pltpu.sync_copy(aBs[f].at[sl], bufB)

        def _cb(i, _):
            ix = pl.ds(i * _c16, 16)
            bufO[ix] = (bufA[ix] + bufB[ix]) * bufW[ix]
            bufA[ix] = jnp.zeros((16,), _f32)
            return np.int32(0)
        lax.fori_loop(np.int32(0), np.int32(SL // 16), _cb, np.int32(0))

        pltpu.sync_copy(bufO, zS[f].at[sl])

        @pl.when(c == 0)
        def _():
            pltpu.sync_copy(bufO, aS[f].at[sl])

        @pl.when(c == 1)
        def _():
            pltpu.sync_copy(bufA, aS[f].at[sl])

    plsc.subcore_barrier()

    def _idx(off, ir, ic):
        pltpu.sync_copy(row_hbm.at[pl.ds(off, CH)], ir)
        pltpu.sync_copy(col_hbm.at[pl.ds(off, CH)], ic)

    def _gath(ir, g0, g1, g2):
        pltpu.async_copy(zS0.at[ir], g0, sem)
        pltpu.async_copy(zS1.at[ir], g1, sem)
        pltpu.async_copy(zS2.at[ir], g2, sem)

    def _gwait(ir, g0, g1, g2):
        pltpu.make_async_copy(zS0.at[ir], g0, sem).wait()
        pltpu.make_async_copy(zS1.at[ir], g1, sem).wait()
        pltpu.make_async_copy(zS2.at[ir], g2, sem).wait()

    def _scat(ic, g0, g1, g2):
        pltpu.sync_copy(g0, aS0.at[ic], add=True)
        pltpu.sync_copy(g1, aS1.at[ic], add=True)
        pltpu.sync_copy(g2, aS2.at[ic], add=True)

    base = wid * _cEPW

    def _chunk(k, _):
        off = base + k * _cCH
        _idx(off, rowv0, colv0)
        _gath(rowv0, gva0, gva1, gva2)
        _gwait(rowv0, gva0, gva1, gva2)
        _scat(colv0, gva0, gva1, gva2)
        return np.int32(0)
    lax.fori_loop(np.int32(0), np.int32(NFULL), _chunk, np.int32(0))
    toff = base + NFULL * _cCH
    pltpu.sync_copy(row_hbm.at[pl.ds(toff, CTAIL)], rowt)
    pltpu.sync_copy(col_hbm.at[pl.ds(toff, CTAIL)], colt)
    pltpu.async_copy(zS0.at[rowt], gt0, sem)
    pltpu.async_copy(zS1.at[rowt], gt1, sem)
    pltpu.async_copy(zS2.at[rowt], gt2, sem)
    pltpu.make_async_copy(zS0.at[rowt], gt0, sem).wait()
    pltpu.make_async_copy(zS1.at[rowt], gt1, sem).wait()
    pltpu.make_async_copy(zS2.at[rowt], gt2, sem).wait()
    pltpu.sync_copy(gt0, aS0.at[colt], add=True)
    pltpu.sync_copy(gt1, aS1.at[colt], add=True)
    pltpu.sync_copy(gt2, aS2.at[colt], add=True)

    plsc.subcore_barrier()

    for f in range(3):
        @pl.when(c == 0)
        def _():
            pltpu.sync_copy(aS[f].at[sl], oAs[f].at[sl])

        @pl.when(c == 1)
        def _():
            pltpu.sync_copy(aS[f].at[sl], oBs[f].at[sl])


@functools.cache
def _hop_kernel():
  return pl.kernel(
    _hop_body,
    out_type=tuple(
        jax.ShapeDtypeStruct((NPAD,), _f32) for _ in range(6)),
    mesh=_mesh(),
    scratch_types=(
        pltpu.VMEM((CH,), _i32),
        pltpu.VMEM((CH,), _i32),
        pltpu.VMEM((CTAIL,), _i32),
        pltpu.VMEM((CTAIL,), _i32),
        pltpu.VMEM((CH,), _f32),
        pltpu.VMEM((CH,), _f32),
        pltpu.VMEM((CH,), _f32),
        pltpu.VMEM((CTAIL,), _f32),
        pltpu.VMEM((CTAIL,), _f32),
        pltpu.VMEM((CTAIL,), _f32),
        pltpu.VMEM((SL,), _f32),
        pltpu.VMEM((SL,), _f32),
        pltpu.VMEM((SL,), _f32),
        pltpu.VMEM((SL,), _f32),
        pltpu.VMEM_SHARED((NPAD,), _f32),
        pltpu.VMEM_SHARED((NPAD,), _f32),
        pltpu.VMEM_SHARED((NPAD,), _f32),
        pltpu.VMEM_SHARED((NPAD,), _f32),
        pltpu.VMEM_SHARED((NPAD,), _f32),
        pltpu.VMEM_SHARED((NPAD,), _f32),
        pltpu.SemaphoreType.DMA,
    ),
  )


TB = 2048
NTB = NPAD // TB


def _prep_body(degA, degB, xT, w_o, dinv_o, zinit_o):
    deg = degA[...] + degB[...] + 1.0
    sq = jnp.sqrt(deg)
    dinv = 1.0 / sq
    w_o[...] = dinv * dinv
    dinv_o[...] = dinv
    zinit_o[...] = xT[...] * sq


_prep_kernel = pl.pallas_call(
    _prep_body,
    grid=(NTB,),
    in_specs=[
        pl.BlockSpec((1, TB), lambda t: (0, t)),
        pl.BlockSpec((1, TB), lambda t: (0, t)),
        pl.BlockSpec((3, TB), lambda t: (0, t)),
    ],
    out_specs=[
        pl.BlockSpec((1, TB), lambda t: (0, t)),
        pl.BlockSpec((1, TB), lambda t: (0, t)),
        pl.BlockSpec((3, TB), lambda t: (0, t)),
    ],
    out_shape=[
        jax.ShapeDtypeStruct((1, NPAD), _f32),
        jax.ShapeDtypeStruct((1, NPAD), _f32),
        jax.ShapeDtypeStruct((3, NPAD), _f32),
    ],
)


def _stats_body(aA, aB, dinv, wsgcT, bn1g, bn1b, p_o, wp_o, acc):
    t = pl.program_id(0)

    @pl.when(t == 0)
    def _():
        for k in range(10):
            acc[k] = 0.0

    p = dinv[...] * (aA[...] + aB[...])
    p_o[...] = p
    idx = 0
    for f in range(3):
        acc[idx] += jnp.sum(p[f:f + 1, :])
        idx += 1
    for f in range(3):
        for g in range(f, 3):
            acc[idx] += jnp.sum(p[f:f + 1, :] * p[g:g + 1, :])
            idx += 1

    @pl.when(t == NTB - 1)
    def _():
        n = jnp.float32(N)
        mu = [acc[f] / n for f in range(3)]
        raw = {}
        idx2 = 3
        for f in range(3):
            for g in range(f, 3):
                raw[(f, g)] = acc[idx2] / n
                idx2 += 1
        WT = wsgcT[...]
        v = jnp.zeros((H1, 1), _f32)
        wmu = jnp.zeros((H1, 1), _f32)
        for f in range(3):
            wmu = wmu + WT[:, f:f + 1] * mu[f]
            for g in range(3):
                cov = raw[(min(f, g), max(f, g))] - mu[f] * mu[g]
                v = v + (WT[:, f:f + 1] * WT[:, g:g + 1]) * cov
        gam = bn1g[...] * lax.rsqrt(v + EPS)
        bprime = bn1b[...] - gam * wmu
        wp_o[...] = jnp.concatenate(
            [WT * gam, bprime, jnp.zeros((H1, 4), _f32)], axis=1)


_stats_kernel = pl.pallas_call(
    _stats_body,
    grid=(NTB,),
    in_specs=[
        pl.BlockSpec((3, TB), lambda t: (0, t)),
        pl.BlockSpec((3, TB), lambda t: (0, t)),
        pl.BlockSpec((1, TB), lambda t: (0, t)),
        pl.BlockSpec((H1, 3), lambda t: (0, 0)),
        pl.BlockSpec((H1, 1), lambda t: (0, 0)),
        pl.BlockSpec((H1, 1), lambda t: (0, 0)),
    ],
    out_specs=[
        pl.BlockSpec((3, TB), lambda t: (0, t)),
        pl.BlockSpec((H1, 8), lambda t: (0, 0)),
    ],
    out_shape=[
        jax.ShapeDtypeStruct((3, NPAD), _f32),
        jax.ShapeDtypeStruct((H1, 8), _f32),
    ],
    scratch_shapes=[pltpu.SMEM((16,), _f32)],
)


TE = 2048
NTE = NPAD // TE
NEG = float("-inf")


def _head_body(p, bt, wp, w1T, b1c, bn2g, bn2b, w2T, b2c, bn3g, bn3b,
               w3T, b3c, cntA, cntB, o_ref, pool):
    t = pl.program_id(0)

    @pl.when(t == 0)
    def _():
        pool[...] = jnp.full((H1, B), NEG, _f32)

    @pl.when(t < NTE)
    def _():
        wpv = wp[...]
        pv = p[...]
        zT = (wpv[:, 0:1] * pv[0:1, :] + wpv[:, 1:2] * pv[1:2, :]
              + wpv[:, 2:3] * pv[2:3, :])
        b = bt[...]
        lo = jnp.min(b)
        hi = jnp.minimum(jnp.max(b), B - 1)
        lane = lax.broadcasted_iota(_i32, (H1, B), 1)

        def _seg(sg, _):
            m = jnp.max(jnp.where(b == sg, zT, NEG), axis=1,
                        keepdims=True)
            cur = pool[...]
            pool[...] = jnp.where(lane == sg, jnp.maximum(cur, m), cur)
            return np.int32(0)
        lax.fori_loop(lo, hi + 1, _seg, np.int32(0))

    @pl.when(t == NTE)
    def _():
        cnt = (cntA[...] + cntB[...])[:, :B]
        pooled = pool[...] + wp[...][:, 3:4]
        pooled = jnp.where(cnt > 0.0, jnp.maximum(pooled, 0.0), NEG)

        h = jnp.dot(w1T[...], pooled, precision=lax.Precision.HIGHEST,
                    preferred_element_type=_f32) + b1c[...]
        m = jnp.mean(h, axis=1, keepdims=True)
        v = jnp.mean((h - m) ** 2, axis=1, keepdims=True)
        h = (h - m) * lax.rsqrt(v + EPS) * bn2g[...] + bn2b[...]
        h = jnp.maximum(h, 0.0)

        h = jnp.dot(w2T[...], h, precision=lax.Precision.HIGHEST,
                    preferred_element_type=_f32) + b2c[...]
        m = jnp.mean(h, axis=1, keepdims=True)
        v = jnp.mean((h - m) ** 2, axis=1, keepdims=True)
        h = (h - m) * lax.rsqrt(v + EPS) * bn3g[...] + bn3b[...]
        h = jnp.maximum(h, 0.0)

        h = jnp.dot(w3T[...], h, precision=lax.Precision.HIGHEST,
                    preferred_element_type=_f32) + b3c[...]
        mx = jnp.max(h, axis=0, keepdims=True)
        lse = jnp.log(jnp.sum(jnp.exp(h - mx), axis=0, keepdims=True)) + mx
        o_ref[...] = h - lse


_head_kernel = pl.pallas_call(
    _head_body,
    grid=(NTE + 1,),
    in_specs=[
        pl.BlockSpec((3, TE), lambda t: (0, jnp.minimum(t, NTE - 1))),
        pl.BlockSpec((1, TE), lambda t: (0, jnp.minimum(t, NTE - 1))),
        pl.BlockSpec((H1, 8), lambda t: (0, 0)),
        pl.BlockSpec((H2, H1), lambda t: (0, 0)),
        pl.BlockSpec((H2, 1), lambda t: (0, 0)),
        pl.BlockSpec((H2, 1), lambda t: (0, 0)),
        pl.BlockSpec((H2, 1), lambda t: (0, 0)),
        pl.BlockSpec((H3, H2), lambda t: (0, 0)),
        pl.BlockSpec((H3, 1), lambda t: (0, 0)),
        pl.BlockSpec((H3, 1), lambda t: (0, 0)),
        pl.BlockSpec((H3, 1), lambda t: (0, 0)),
        pl.BlockSpec((CLS, H3), lambda t: (0, 0)),
        pl.BlockSpec((CLS, 1), lambda t: (0, 0)),
        pl.BlockSpec((1, 128), lambda t: (0, 0)),
        pl.BlockSpec((1, 128), lambda t: (0, 0)),
    ],
    out_specs=pl.BlockSpec((CLS, B), lambda t: (0, 0)),
    out_shape=jax.ShapeDtypeStruct((CLS, B), _f32),
    scratch_shapes=[pltpu.VMEM((H1, B), _f32)],
)


def kernel(x, edge_index, batch, W_sgc, b_sgc, bn1_g, bn1_b, W1, b1,
           bn2_g, bn2_b, W2, b2, bn3_g, bn3_b, W3, b3):
    with jax.enable_x64(False):
        out = _kernel_impl(x, edge_index, batch, W_sgc, b_sgc, bn1_g,
                           bn1_b, W1, b1, bn2_g, bn2_b, W2, b2, bn3_g,
                           bn3_b, W3, b3)
    return out.astype(jnp.float64)


def _kernel_impl(x, edge_index, batch, W_sgc, b_sgc, bn1_g, bn1_b, W1, b1,
                 bn2_g, bn2_b, W2, b2, bn3_g, bn3_b, W3, b3):
    ei = edge_index.astype(_i32)
    npad = NPAD - N
    row = ei[0]
    col = ei[1]

    b32 = batch.astype(_i32)
    batch_sc = jnp.concatenate([b32, jnp.full((npad,), 64, _i32)])
    batch_tc = jnp.concatenate(
        [b32, jnp.full((npad,), 9999, _i32)]).reshape(1, NPAD)

    xT = jnp.pad(x.astype(_f32).T, ((0, 0), (0, npad)))

    degA, degB, cntA, cntB = _deg_kernel()(col, batch_sc)
    w1d, dinv1d, zinit = _prep_kernel(
        degA.reshape(1, NPAD), degB.reshape(1, NPAD), xT)
    w_flat = w1d.reshape(NPAD)

    z1 = jnp.zeros((NPAD,), _f32)
    st = [zinit[0], zinit[1], zinit[2], z1, z1, z1]
    for _ in range(K):
        st = list(_hop_kernel()(row, col, w_flat, *st))
    aA = jnp.stack(st[0:3])
    aB = jnp.stack(st[3:6])

    wsgcT = W_sgc.astype(_f32).T
    bn1g2 = (bn1_g.astype(_f32) * jnp.ones((1,), _f32)).reshape(H1, 1)
    bn1b2 = bn1_b.astype(_f32).reshape(H1, 1)
    p3, wp = _stats_kernel(aA, aB, dinv1d, wsgcT, bn1g2, bn1b2)

    oT = _head_kernel(
        p3, batch_tc, wp,
        W1.astype(_f32).T, b1.astype(_f32).reshape(H2, 1),
        bn2_g.astype(_f32).reshape(H2, 1), bn2_b.astype(_f32).reshape(H2, 1),
        W2.astype(_f32).T, b2.astype(_f32).reshape(H3, 1),
        bn3_g.astype(_f32).reshape(H3, 1), bn3_b.astype(_f32).reshape(H3, 1),
        W3.astype(_f32).T, b3.astype(_f32).reshape(CLS, 1),
        cntA.reshape(1, 128), cntB.reshape(1, 128),
    )
    return oT.T
